# baseline (speedup 1.0000x reference)
import jax
import jax.numpy as jnp
from jax import lax
from jax.experimental import pallas as pl
from jax.experimental.pallas import tpu as pltpu
from jax.experimental.pallas import tpu_sc as plsc

GRID = 32
K3 = GRID * GRID * GRID
N_PTS = 320000
C_FULL = 128
CH = 32
NC, NS = 2, 16
SUB = 128
NROWS = N_PTS // SUB
NB = 7
D = 3
ROWS_PER_TILE = K3 // NS


def _pidx_body(pt_ref, o_ref):
    x = pt_ref[0, :]
    y = pt_ref[1, :]
    z = pt_ref[2, :]
    ix = ((x + 1.0) * 16.0).astype(jnp.int32)
    iy = ((y + 1.0) * 16.0).astype(jnp.int32)
    iz = ((z + 1.0) * 16.0).astype(jnp.int32)
    pidx = ix * 1024 + iy * 32 + iz
    o_ref[...] = pidx.reshape(o_ref.shape)


def _pidx(points_t):
    return pl.pallas_call(
        _pidx_body,
        out_shape=jax.ShapeDtypeStruct((NROWS, SUB), jnp.int32),
    )(points_t)


def _sc_body(pf_ref, pidx_ref, za_ref,
             sums_ref, hist_out_ref,
             idx_v, hist_v, rows_v, sums_sh, gsem, ssem, psem):
    c = lax.axis_index("c")
    s = lax.axis_index("s")
    is_c0 = c == 0
    r0 = s * ROWS_PER_TILE
    rlo = s * NROWS // NS
    nsub = (s + 1) * NROWS // NS - rlo

    zero_f = jnp.zeros((16,), jnp.float32)
    ones_f = jnp.ones((16,), jnp.float32)

    @pl.when(is_c0)
    def _():
        def hz(k, carry):
            for u in range(4):
                hist_v[pl.ds(k * 64 + u * 16, 16)] = zero_f
            return carry

        lax.fori_loop(0, K3 // 64, hz, 0)

    def slot(j):
        return j % NB

    def i_src(j):
        return pidx_ref.at[rlo + j]

    def i_dst(j):
        return idx_v.at[slot(j)]

    def g_src(ch0, j):
        return pf_ref.at[pl.ds((rlo + j) * SUB, SUB), pl.ds(ch0, CH)]

    def g_dst(j):
        return rows_v.at[pl.ds(slot(j) * SUB, SUB)]

    def s_dst(j):
        return sums_sh.at[idx_v.at[slot(j)]]

    def stage_in(ch0, j):
        pltpu.async_copy(i_src(j), i_dst(j), psem)
        pltpu.async_copy(g_src(ch0, j), g_dst(j), gsem)

    def issue_scatter(j):
        pltpu.async_copy(g_dst(j), s_dst(j), ssem, add=True)

    def drain_one_scatter():
        pltpu.make_async_copy(g_dst(0), s_dst(0), ssem).wait()

    def one_pass(p, carry):
        ch0 = c * 64 + p * CH
        pltpu.sync_copy(za_ref, sums_sh.at[pl.ds(r0, ROWS_PER_TILE)])
        plsc.subcore_barrier()

        def prime(i, carry):
            stage_in(ch0, i)
            return carry

        lax.fori_loop(0, NB, prime, 0)

        def step(j, carry):
            @pl.when(j < nsub)
            def _():
                pltpu.make_async_copy(i_src(j), i_dst(j), psem).wait()
                sl = slot(j)

                @pl.when(jnp.logical_and(is_c0, p == 0))
                def _():
                    for u in range(SUB // 16):
                        pidx = idx_v[sl, pl.ds(u * 16, 16)]
                        plsc.addupdate_scatter(hist_v, [pidx], ones_f)

                pltpu.make_async_copy(g_src(ch0, j), g_dst(j), gsem).wait()
                issue_scatter(j)

            @pl.when(j >= D)
            def _():
                drain_one_scatter()

                @pl.when(j - D + NB < nsub)
                def _():
                    stage_in(ch0, j - D + NB)
            return carry

        lax.fori_loop(0, nsub + D, step, 0)

        plsc.subcore_barrier()

        pltpu.sync_copy(sums_sh.at[pl.ds(r0, ROWS_PER_TILE)],
                        sums_ref.at[pl.ds(r0, ROWS_PER_TILE), pl.ds(ch0, CH)])
        return carry

    lax.fori_loop(0, 2, one_pass, 0)

    @pl.when(is_c0)
    def _():
        pltpu.sync_copy(hist_v, hist_out_ref.at[s])


def _sc_scatter(point_feat, pidx):
    za = jnp.zeros((ROWS_PER_TILE, CH), jnp.float32)
    f32 = jnp.float32
    run = pl.kernel(
        _sc_body,
        out_type=(
            jax.ShapeDtypeStruct((K3, C_FULL), f32),
            jax.ShapeDtypeStruct((NS, K3), f32),
        ),
        mesh=plsc.VectorSubcoreMesh(core_axis_name="c", subcore_axis_name="s"),
        scratch_types=[
            pltpu.VMEM((NB, SUB), jnp.int32),
            pltpu.VMEM((K3,), f32),
            pltpu.VMEM((NB * SUB, CH), f32),
            pltpu.VMEM_SHARED((K3, CH), f32),
            pltpu.SemaphoreType.DMA,
            pltpu.SemaphoreType.DMA,
            pltpu.SemaphoreType.DMA,
        ],
        compiler_params=pltpu.CompilerParams(
            use_tc_tiling_on_sc=False, needs_layout_passes=False),
    )
    return run(point_feat, pidx, za)


def _div_body(s_ref, h_ref, o_ref):
    counts = jnp.sum(h_ref[...], axis=0)
    inv = 1.0 / jnp.maximum(counts, 1.0)
    o_ref[...] = s_ref[...] * inv[:, None]


def _divide(sums, hists):
    blk = 2048
    return pl.pallas_call(
        _div_body,
        grid=(K3 // blk,),
        in_specs=[pl.BlockSpec((blk, C_FULL), lambda i: (i, 0)),
                  pl.BlockSpec((NS, blk), lambda i: (0, i))],
        out_specs=pl.BlockSpec((blk, C_FULL), lambda i: (i, 0)),
        out_shape=jax.ShapeDtypeStruct((K3, C_FULL), jnp.float32),
    )(sums, hists)


def kernel(point_feat, points):
    pidx = _pidx(points.T)
    sums, hists = _sc_scatter(point_feat, pidx)
    out = _divide(sums, hists)
    return out.reshape(GRID, GRID, GRID, C_FULL)


# speedup vs baseline: 1.0415x; 1.0415x over previous
import jax
import jax.numpy as jnp
from jax import lax
from jax.experimental import pallas as pl
from jax.experimental.pallas import tpu as pltpu
from jax.experimental.pallas import tpu_sc as plsc

GRID = 32
K3 = GRID * GRID * GRID
N_PTS = 320000
C_FULL = 128
CH = 32
NC, NS = 2, 16
SUB = 128
NROWS = N_PTS // SUB
NB = 7
D = 2
ROWS_PER_TILE = K3 // NS


def _pidx_body(pt_ref, o_ref):
    x = pt_ref[0, :]
    y = pt_ref[1, :]
    z = pt_ref[2, :]
    ix = ((x + 1.0) * 16.0).astype(jnp.int32)
    iy = ((y + 1.0) * 16.0).astype(jnp.int32)
    iz = ((z + 1.0) * 16.0).astype(jnp.int32)
    pidx = ix * 1024 + iy * 32 + iz
    o_ref[...] = pidx.reshape(o_ref.shape)


def _pidx(points_t):
    return pl.pallas_call(
        _pidx_body,
        out_shape=jax.ShapeDtypeStruct((NROWS, SUB), jnp.int32),
    )(points_t)


def _sc_body(pf_ref, pidx_ref, za_ref,
             sums_ref, hist_out_ref,
             idx_v, hist_v, rows_v, sums_sh, gsem, ssem, psem):
    c = lax.axis_index("c")
    s = lax.axis_index("s")
    is_c0 = c == 0
    r0 = s * ROWS_PER_TILE
    rlo = s * NROWS // NS
    nsub = (s + 1) * NROWS // NS - rlo

    zero_f = jnp.zeros((16,), jnp.float32)
    ones_f = jnp.ones((16,), jnp.float32)

    @pl.when(is_c0)
    def _():
        def hz(k, carry):
            for u in range(4):
                hist_v[pl.ds(k * 64 + u * 16, 16)] = zero_f
            return carry

        lax.fori_loop(0, K3 // 64, hz, 0)

    def slot(j):
        return j % NB

    def i_src(j):
        return pidx_ref.at[rlo + j]

    def i_dst(j):
        return idx_v.at[slot(j)]

    def g_src(ch0, j):
        return pf_ref.at[pl.ds((rlo + j) * SUB, SUB), pl.ds(ch0, CH)]

    def g_dst(j):
        return rows_v.at[pl.ds(slot(j) * SUB, SUB)]

    def s_dst(j):
        return sums_sh.at[idx_v.at[slot(j)]]

    def stage_in(ch0, j):
        pltpu.async_copy(i_src(j), i_dst(j), psem)
        pltpu.async_copy(g_src(ch0, j), g_dst(j), gsem)

    def issue_scatter(j):
        pltpu.async_copy(g_dst(j), s_dst(j), ssem, add=True)

    def drain_one_scatter():
        pltpu.make_async_copy(g_dst(0), s_dst(0), ssem).wait()

    def one_pass(p, carry):
        ch0 = c * 64 + p * CH
        pltpu.sync_copy(za_ref, sums_sh.at[pl.ds(r0, ROWS_PER_TILE)])
        plsc.subcore_barrier()

        def prime(i, carry):
            stage_in(ch0, i)
            return carry

        lax.fori_loop(0, NB, prime, 0)

        def step(j, carry):
            @pl.when(j < nsub)
            def _():
                pltpu.make_async_copy(i_src(j), i_dst(j), psem).wait()
                sl = slot(j)

                @pl.when(jnp.logical_and(is_c0, p == 0))
                def _():
                    for u in range(SUB // 16):
                        pidx = idx_v[sl, pl.ds(u * 16, 16)]
                        plsc.addupdate_scatter(hist_v, [pidx], ones_f)

                pltpu.make_async_copy(g_src(ch0, j), g_dst(j), gsem).wait()
                issue_scatter(j)

            @pl.when(j >= D)
            def _():
                drain_one_scatter()

                @pl.when(j - D + NB < nsub)
                def _():
                    stage_in(ch0, j - D + NB)
            return carry

        lax.fori_loop(0, nsub + D, step, 0)

        plsc.subcore_barrier()

        pltpu.sync_copy(sums_sh.at[pl.ds(r0, ROWS_PER_TILE)],
                        sums_ref.at[pl.ds(r0, ROWS_PER_TILE), pl.ds(ch0, CH)])
        return carry

    lax.fori_loop(0, 2, one_pass, 0)

    @pl.when(is_c0)
    def _():
        pltpu.sync_copy(hist_v, hist_out_ref.at[s])


def _sc_scatter(point_feat, pidx):
    za = jnp.zeros((ROWS_PER_TILE, CH), jnp.float32)
    f32 = jnp.float32
    run = pl.kernel(
        _sc_body,
        out_type=(
            jax.ShapeDtypeStruct((K3, C_FULL), f32),
            jax.ShapeDtypeStruct((NS, K3), f32),
        ),
        mesh=plsc.VectorSubcoreMesh(core_axis_name="c", subcore_axis_name="s"),
        scratch_types=[
            pltpu.VMEM((NB, SUB), jnp.int32),
            pltpu.VMEM((K3,), f32),
            pltpu.VMEM((NB * SUB, CH), f32),
            pltpu.VMEM_SHARED((K3, CH), f32),
            pltpu.SemaphoreType.DMA,
            pltpu.SemaphoreType.DMA,
            pltpu.SemaphoreType.DMA,
        ],
        compiler_params=pltpu.CompilerParams(
            use_tc_tiling_on_sc=False, needs_layout_passes=False),
    )
    return run(point_feat, pidx, za)


def _div_body(s_ref, h_ref, o_ref):
    counts = jnp.sum(h_ref[...], axis=0)
    inv = 1.0 / jnp.maximum(counts, 1.0)
    o_ref[...] = s_ref[...] * inv[:, None]


def _divide(sums, hists):
    blk = 4096
    return pl.pallas_call(
        _div_body,
        grid=(K3 // blk,),
        in_specs=[pl.BlockSpec((blk, C_FULL), lambda i: (i, 0)),
                  pl.BlockSpec((NS, blk), lambda i: (0, i))],
        out_specs=pl.BlockSpec((blk, C_FULL), lambda i: (i, 0)),
        out_shape=jax.ShapeDtypeStruct((K3, C_FULL), jnp.float32),
    )(sums, hists)


def kernel(point_feat, points):
    pidx = _pidx(points.T)
    sums, hists = _sc_scatter(point_feat, pidx)
    out = _divide(sums, hists)
    return out.reshape(GRID, GRID, GRID, C_FULL)


# speedup vs baseline: 1.0511x; 1.0092x over previous
import jax
import jax.numpy as jnp
from jax import lax
from jax.experimental import pallas as pl
from jax.experimental.pallas import tpu as pltpu
from jax.experimental.pallas import tpu_sc as plsc

GRID = 32
K3 = GRID * GRID * GRID
N_PTS = 320000
C_FULL = 128
CH = 32
NC, NS = 2, 16
SUB = 128
NROWS = N_PTS // SUB
NB = 7
D = 2
ROWS_PER_TILE = K3 // NS


def _pidx_body(pt_ref, o_ref):
    x = pt_ref[0, :]
    y = pt_ref[1, :]
    z = pt_ref[2, :]
    ix = ((x + 1.0) * 16.0).astype(jnp.int32)
    iy = ((y + 1.0) * 16.0).astype(jnp.int32)
    iz = ((z + 1.0) * 16.0).astype(jnp.int32)
    pidx = ix * 1024 + iy * 32 + iz
    o_ref[...] = pidx.reshape(o_ref.shape)


def _pidx(points_t):
    return pl.pallas_call(
        _pidx_body,
        out_shape=jax.ShapeDtypeStruct((NROWS, SUB), jnp.int32),
    )(points_t)


def _sc_body(pf_ref, pidx_ref, za_ref,
             sums_ref, hist_out_ref,
             idx_v, hist_v, rows_v, sums_sh, gsem, ssem, psem):
    c = lax.axis_index("c")
    s = lax.axis_index("s")
    is_c0 = c == 0
    r0 = s * ROWS_PER_TILE
    rlo = s * NROWS // NS
    nsub = (s + 1) * NROWS // NS - rlo

    zero_f = jnp.zeros((16,), jnp.float32)
    ones_f = jnp.ones((16,), jnp.float32)

    @pl.when(is_c0)
    def _():
        def hz(k, carry):
            for u in range(4):
                hist_v[pl.ds(k * 64 + u * 16, 16)] = zero_f
            return carry

        lax.fori_loop(0, K3 // 64, hz, 0)

    def slot(j):
        return j % NB

    def i_src(j):
        return pidx_ref.at[rlo + j]

    def i_dst(j):
        return idx_v.at[slot(j)]

    def g_src(ch0, j):
        return pf_ref.at[pl.ds((rlo + j) * SUB, SUB), pl.ds(ch0, CH)]

    def g_dst(j):
        return rows_v.at[pl.ds(slot(j) * SUB, SUB)]

    def s_dst(j):
        return sums_sh.at[idx_v.at[slot(j)]]

    def stage_in(ch0, j):
        pltpu.async_copy(i_src(j), i_dst(j), psem)
        pltpu.async_copy(g_src(ch0, j), g_dst(j), gsem)

    def issue_scatter(j):
        pltpu.async_copy(g_dst(j), s_dst(j), ssem, add=True)

    def drain_one_scatter():
        pltpu.make_async_copy(g_dst(0), s_dst(0), ssem).wait()

    def one_pass(p, carry):
        ch0 = c * 64 + p * CH
        pltpu.sync_copy(za_ref, sums_sh.at[pl.ds(r0, ROWS_PER_TILE)])
        plsc.subcore_barrier()

        def prime(i, carry):
            stage_in(ch0, i)
            return carry

        lax.fori_loop(0, NB, prime, 0)

        def step(j, carry):
            @pl.when(j < nsub)
            def _():
                pltpu.make_async_copy(i_src(j), i_dst(j), psem).wait()
                sl = slot(j)

                @pl.when(jnp.logical_and(is_c0, p == 0))
                def _():
                    for u in range(SUB // 16):
                        pidx = idx_v[sl, pl.ds(u * 16, 16)]
                        plsc.addupdate_scatter(hist_v, [pidx], ones_f)

                pltpu.make_async_copy(g_src(ch0, j), g_dst(j), gsem).wait()
                issue_scatter(j)

            @pl.when(j >= D)
            def _():
                drain_one_scatter()

                @pl.when(j - D + NB < nsub)
                def _():
                    stage_in(ch0, j - D + NB)
            return carry

        lax.fori_loop(0, nsub + D, step, 0)

        plsc.subcore_barrier()

        pltpu.sync_copy(sums_sh.at[pl.ds(r0, ROWS_PER_TILE)],
                        sums_ref.at[pl.ds(r0, ROWS_PER_TILE), pl.ds(ch0, CH)])
        return carry

    lax.fori_loop(0, 2, one_pass, 0)

    @pl.when(is_c0)
    def _():
        pltpu.sync_copy(hist_v, hist_out_ref.at[s])


def _sc_scatter(point_feat, pidx):
    za = jnp.zeros((ROWS_PER_TILE, CH), jnp.float32)
    f32 = jnp.float32
    run = pl.kernel(
        _sc_body,
        out_type=(
            jax.ShapeDtypeStruct((K3, C_FULL), f32),
            jax.ShapeDtypeStruct((NS, K3), f32),
        ),
        mesh=plsc.VectorSubcoreMesh(core_axis_name="c", subcore_axis_name="s"),
        scratch_types=[
            pltpu.VMEM((NB, SUB), jnp.int32),
            pltpu.VMEM((K3,), f32),
            pltpu.VMEM((NB * SUB, CH), f32),
            pltpu.VMEM_SHARED((K3, CH), f32),
            pltpu.SemaphoreType.DMA,
            pltpu.SemaphoreType.DMA,
            pltpu.SemaphoreType.DMA,
        ],
        compiler_params=pltpu.CompilerParams(
            use_tc_tiling_on_sc=False, needs_layout_passes=False),
    )
    return run(point_feat, pidx, za)


def _div_body(s_ref, h_ref, o_ref):
    counts = jnp.sum(h_ref[...], axis=0)
    inv = 1.0 / jnp.maximum(counts, 1.0)
    o_ref[...] = s_ref[...] * inv[:, None]


def _divide(sums, hists):
    blk = 8192
    return pl.pallas_call(
        _div_body,
        grid=(K3 // blk,),
        in_specs=[pl.BlockSpec((blk, C_FULL), lambda i: (i, 0)),
                  pl.BlockSpec((NS, blk), lambda i: (0, i))],
        out_specs=pl.BlockSpec((blk, C_FULL), lambda i: (i, 0)),
        out_shape=jax.ShapeDtypeStruct((K3, C_FULL), jnp.float32),
    )(sums, hists)


def kernel(point_feat, points):
    pidx = _pidx(points.T)
    sums, hists = _sc_scatter(point_feat, pidx)
    out = _divide(sums, hists)
    return out.reshape(GRID, GRID, GRID, C_FULL)
